# Initial kernel scaffold; baseline (speedup 1.0000x reference)
import jax
import jax.numpy as jnp
from jax.experimental import pallas as pl


def kernel(x, edge_index, W1, a_src1, a_dst1, b1, W2, a_src2, a_dst2, b2):
    raise NotImplementedError("write your pallas kernel here")


# speedup vs baseline: 20.9713x; 20.9713x over previous
import functools

import jax
import jax.numpy as jnp
from jax import lax
from jax.experimental import pallas as pl
from jax.experimental.pallas import tpu as pltpu
from jax.experimental.pallas import tpu_sc as plsc

N = 10000
D_IN = 128
H1 = 4
DH = 64
H2 = 1
DO = 128
E = 320000

NP = 10240
DUMMY = N
NC = 2
NS = 16
L = 16
K = 128
E_TOT = E + N
NW = NC * NS
EW = -(-E_TOT // (NW * L)) * L
E_PAD = NW * EW
EWP = EW + K
RW = NW * EWP
NH = NP // 2
NHP = NH + K
ZRT = NHP // NS
WOT = NH // NS
BM = 256
DEN_W = 16

_f32 = jnp.float32
_i32 = jnp.int32


def _tc1_body(x_ref, w_ref, a_ref, h_ref, ab_ref):
    h = jnp.dot(x_ref[...], w_ref[...], preferred_element_type=_f32)
    h_ref[0] = h[:, : 2 * DH]
    h_ref[1] = h[:, 2 * DH:]
    ab_ref[...] = lax.dot_general(a_ref[...], h, (((0,), (1,)), ((), ())),
                                  preferred_element_type=_f32)


_tc1 = pl.pallas_call(
    _tc1_body,
    grid=(NP // BM,),
    in_specs=[pl.BlockSpec((BM, D_IN), lambda m: (m, 0)),
              pl.BlockSpec((D_IN, H1 * DH), lambda m: (0, 0)),
              pl.BlockSpec((H1 * DH, 2 * H1), lambda m: (0, 0))],
    out_specs=[pl.BlockSpec((2, BM, 2 * DH), lambda m: (0, m, 0)),
               pl.BlockSpec((2 * H1, BM), lambda m: (0, m))],
    out_shape=[jax.ShapeDtypeStruct((2, NP, 2 * DH), _f32),
               jax.ShapeDtypeStruct((2 * H1, NP), _f32)],
)


def _tc2_body(acc_ref, den_ref, b1_ref, w2_ref, a2_ref, h2_ref, ab2_ref):
    eps = _f32(1e-16)
    parts = []
    for c in range(2):
        a = acc_ref[c]
        d = den_ref[c]
        for hd in range(2):
            num = a[:, hd * DH:(hd + 1) * DH]
            dd = jnp.broadcast_to(d[:, hd:hd + 1], (BM, DH))
            parts.append(num / (dd + eps))
    h1n = jnp.concatenate(parts, axis=1) + b1_ref[...]
    h1n = jnp.maximum(h1n, 0.0)
    h2 = jnp.dot(h1n, w2_ref[...], preferred_element_type=_f32)
    h2_ref[0] = h2[:, : DO // 2]
    h2_ref[1] = h2[:, DO // 2:]
    ab2_ref[...] = lax.dot_general(a2_ref[...], h2, (((0,), (1,)), ((), ())),
                                   preferred_element_type=_f32)


_tc2 = pl.pallas_call(
    _tc2_body,
    grid=(NP // BM,),
    in_specs=[pl.BlockSpec((2, BM, 2 * DH), lambda m: (0, m, 0)),
              pl.BlockSpec((2, BM, DEN_W), lambda m: (0, m, 0)),
              pl.BlockSpec((1, H1 * DH), lambda m: (0, 0)),
              pl.BlockSpec((H1 * DH, DO), lambda m: (0, 0)),
              pl.BlockSpec((DO, 2), lambda m: (0, 0))],
    out_specs=[pl.BlockSpec((2, BM, DO // 2), lambda m: (0, m, 0)),
               pl.BlockSpec((2, BM), lambda m: (0, m))],
    out_shape=[jax.ShapeDtypeStruct((2, NP, DO // 2), _f32),
               jax.ShapeDtypeStruct((2, NP), _f32)],
)


def _tc3_body(acc_ref, den_ref, b2_ref, o_ref):
    eps = _f32(1e-16)
    parts = []
    for c in range(2):
        a = acc_ref[c]
        d = jnp.broadcast_to(den_ref[c][:, 0:1], (BM, DO // 2))
        parts.append(a / (d + eps))
    o_ref[...] = jnp.concatenate(parts, axis=1) + b2_ref[...]


_tc3 = pl.pallas_call(
    _tc3_body,
    grid=(NP // BM,),
    in_specs=[pl.BlockSpec((2, BM, DO // 2), lambda m: (0, m, 0)),
              pl.BlockSpec((2, BM, DEN_W), lambda m: (0, m, 0)),
              pl.BlockSpec((1, DO), lambda m: (0, 0))],
    out_specs=pl.BlockSpec((BM, DO), lambda m: (m, 0)),
    out_shape=jax.ShapeDtypeStruct((NP, DO), _f32),
)


def _sc_mesh():
    return plsc.VectorSubcoreMesh(core_axis_name="c", subcore_axis_name="s",
                                  num_cores=NC, num_subcores=NS)


def _bucket_body(srcp, dstp, bs0_o, bd0_o, bs1_o, bd1_o, cnt0_o, cnt1_o,
                 in_s, in_d, b0s, b0d, b1s, b1d, cbuf):
    c = lax.axis_index("c")
    s = lax.axis_index("s")
    w = s * NC + c
    pltpu.sync_copy(srcp.at[pl.ds(w * EW, EW)], in_s)
    pltpu.sync_copy(dstp.at[pl.ds(w * EW, EW)], in_d)

    iota = lax.broadcasted_iota(_i32, (L,), 0)

    @pl.loop(0, EW // L, init_carry=(jnp.int32(0), jnp.int32(0)))
    def _part(i, carry):
        cur0, cur1 = carry
        sl = pl.ds(i * L, L)
        sv = in_s[sl]
        dv = in_d[sl]
        m0 = dv < NH
        m1 = jnp.logical_not(m0)
        cs0 = plsc.cumsum(m0.astype(_i32))
        cs1 = (iota + 1) - cs0
        i0 = jnp.where(m0, cur0 + cs0 - 1, 0)
        i1 = jnp.where(m1, cur1 + cs1 - 1, 0)
        plsc.store_scatter(b0s, [i0], sv, mask=m0)
        plsc.store_scatter(b0d, [i0], dv, mask=m0)
        plsc.store_scatter(b1s, [i1], sv, mask=m1)
        plsc.store_scatter(b1d, [i1], dv, mask=m1)
        n0 = lax.reduce_max(cs0, (0,))
        return cur0 + n0, cur1 + (L - n0)

    cur0, cur1 = _part
    for bs, bd, cur, padv in ((b0s, b0d, cur0, NH), (b1s, b1d, cur1, DUMMY)):
        pv = jnp.full((L,), padv, _i32)
        for j in range(K // L):
            plsc.store_scatter(bs, [cur + iota + j * L], pv)
            plsc.store_scatter(bd, [cur + iota + j * L], pv)
    pltpu.sync_copy(b0s, bs0_o.at[pl.ds(w * EWP, EWP)])
    pltpu.sync_copy(b0d, bd0_o.at[pl.ds(w * EWP, EWP)])
    pltpu.sync_copy(b1s, bs1_o.at[pl.ds(w * EWP, EWP)])
    pltpu.sync_copy(b1d, bd1_o.at[pl.ds(w * EWP, EWP)])
    for j in range(128 // L):
        cbuf[pl.ds(j * L, L)] = jnp.broadcast_to(cur0 // K + 1, (L,)).astype(_i32)
    pltpu.sync_copy(cbuf, cnt0_o.at[pl.ds(w * 128, 128)])
    for j in range(128 // L):
        cbuf[pl.ds(j * L, L)] = jnp.broadcast_to(cur1 // K + 1, (L,)).astype(_i32)
    pltpu.sync_copy(cbuf, cnt1_o.at[pl.ds(w * 128, 128)])


@functools.cache
def _bucket_kernel():
    return pl.kernel(
        _bucket_body,
        out_type=[jax.ShapeDtypeStruct((RW,), _i32),
                  jax.ShapeDtypeStruct((RW,), _i32),
                  jax.ShapeDtypeStruct((RW,), _i32),
                  jax.ShapeDtypeStruct((RW,), _i32),
                  jax.ShapeDtypeStruct((NW * 128,), _i32),
                  jax.ShapeDtypeStruct((NW * 128,), _i32)],
        mesh=_sc_mesh(),
        name="gat_edge_bucket",
        compiler_params=pltpu.CompilerParams(needs_layout_passes=False, use_tc_tiling_on_sc=False),
        scratch_types=[
            pltpu.VMEM((EW,), _i32),
            pltpu.VMEM((EW,), _i32),
            pltpu.VMEM((EWP,), _i32),
            pltpu.VMEM((EWP,), _i32),
            pltpu.VMEM((EWP,), _i32),
            pltpu.VMEM((EWP,), _i32),
            pltpu.VMEM((128,), _i32),
        ],
    )


def _make_edge_kernel(D, b):
    Dh = D // 2
    boff = b * NH

    def body(bsrc, bdst, cnts, h2d, absrc, abdst, z, zd, acc_o, den_o,
             asrc_tab, adst_tab, cntbuf, srcbuf, dstbuf, gidx, hbuf,
             exh0, exh1, expay, acc_s, den_s):
        c = lax.axis_index("c")
        s = lax.axis_index("s")
        pltpu.sync_copy(absrc.at[pl.ds(c * (2 * NP), 2 * NP)], asrc_tab)
        pltpu.sync_copy(abdst.at[pl.ds(c * (2 * NP), 2 * NP)], adst_tab)
        pltpu.sync_copy(cnts, cntbuf)
        pltpu.sync_copy(z.at[pl.ds(s * ZRT, ZRT)],
                        acc_s.at[pl.ds(s * ZRT, ZRT)])
        pltpu.sync_copy(zd.at[pl.ds(s * ZRT, ZRT)],
                        den_s.at[pl.ds(s * ZRT, ZRT)])
        plsc.subcore_barrier()

        coff = c * NP

        for r in range(NW // NS):
            w = s * (NW // NS) + r
            ncv = plsc.load_gather(cntbuf, [jnp.full((L,), w * 128, _i32)])
            nch = lax.reduce_max(ncv, (0,))
            base_r = w * EWP

            @pl.loop(0, nch)
            def _chunk(t):
                base = base_r + t * K
                pltpu.sync_copy(bsrc.at[pl.ds(base, K)], srcbuf)
                pltpu.sync_copy(bdst.at[pl.ds(base, K)], dstbuf)

                @pl.loop(0, K // L)
                def _ex(i):
                    sl = pl.ds(i * L, L)
                    sv = srcbuf[sl]
                    dv = dstbuf[sl]
                    gidx[sl] = sv + coff
                    for hd, exh in ((0, exh0), (1, exh1)):
                        off = hd * NP
                        e = (plsc.load_gather(asrc_tab, [sv + off]) +
                             plsc.load_gather(adst_tab, [dv + off]))
                        exh[sl] = jnp.exp(jnp.maximum(e, 0.2 * e))
                    dstbuf[sl] = dv - boff

                pltpu.sync_copy(h2d.at[gidx], hbuf)

                @pl.loop(0, K)
                def _mul(k):
                    kv = jnp.broadcast_to(k, (L,)).astype(_i32)
                    exv0 = plsc.load_gather(exh0, [kv])
                    exv1 = plsc.load_gather(exh1, [kv])
                    lanes = lax.broadcasted_iota(_i32, (L,), 0)
                    expay[k, :] = jnp.where(
                        lanes == 0, exv0,
                        jnp.where(lanes == 1, exv1, jnp.zeros((L,), _f32)))
                    for hd in range(2):
                        exv = exv0 if hd == 0 else exv1
                        for j in range(Dh // L):
                            slc = pl.ds(hd * Dh + j * L, L)
                            hbuf[k, slc] = hbuf[k, slc] * exv

                pltpu.sync_copy(hbuf, acc_s.at[dstbuf], add=True)
                pltpu.sync_copy(expay, den_s.at[dstbuf], add=True)

        plsc.subcore_barrier()
        pltpu.sync_copy(acc_s.at[pl.ds(s * WOT, WOT)],
                        acc_o.at[c, pl.ds(s * WOT, WOT)])
        pltpu.sync_copy(den_s.at[pl.ds(s * WOT, WOT)],
                        den_o.at[c, pl.ds(s * WOT, WOT)])

    return pl.kernel(
        body,
        out_type=[jax.ShapeDtypeStruct((2, NH, D), _f32),
                  jax.ShapeDtypeStruct((2, NH, DEN_W), _f32)],
        mesh=_sc_mesh(),
        name=f"gat_edge_pass_d{D}_b{b}",
        compiler_params=pltpu.CompilerParams(needs_layout_passes=False, use_tc_tiling_on_sc=False),
        scratch_types=[
            pltpu.VMEM((2 * NP,), _f32),
            pltpu.VMEM((2 * NP,), _f32),
            pltpu.VMEM((NW * 128,), _i32),
            pltpu.VMEM((K,), _i32),
            pltpu.VMEM((K,), _i32),
            pltpu.VMEM((K,), _i32),
            pltpu.VMEM((K, D), _f32),
            pltpu.VMEM((K,), _f32),
            pltpu.VMEM((K,), _f32),
            pltpu.VMEM((K, DEN_W), _f32),
            pltpu.VMEM_SHARED((NHP, D), _f32),
            pltpu.VMEM_SHARED((NHP, DEN_W), _f32),
        ],
    )


@functools.cache
def _edge_kernel(D, b):
    return _make_edge_kernel(D, b)


def kernel(x, edge_index, W1, a_src1, a_dst1, b1, W2, a_src2, a_dst2, b2):
    x = x.astype(_f32)
    ei = edge_index.astype(_i32)
    loops = jnp.arange(N, dtype=_i32)
    pad = jnp.full((E_PAD - E_TOT,), DUMMY, _i32)
    srcp = jnp.concatenate([ei[0], loops, pad])
    dstp = jnp.concatenate([ei[1], loops, pad])
    x_pad = jnp.pad(x, ((0, NP - N), (0, 0)))

    eye1 = jnp.eye(H1, dtype=_f32)
    msrc = (eye1[:, None, :] * a_src1[:, :, None]).reshape(H1 * DH, H1)
    mdst = (eye1[:, None, :] * a_dst1[:, :, None]).reshape(H1 * DH, H1)
    a1 = jnp.concatenate([msrc, mdst], axis=1)
    a2 = jnp.stack([a_src2[0], a_dst2[0]], axis=1)
    z1 = jnp.zeros((NP, 2 * DH), _f32)
    z2 = jnp.zeros((NP, DO // 2), _f32)
    zd = jnp.zeros((NP, DEN_W), _f32)

    bs0, bd0, bs1, bd1, cnt0, cnt1 = _bucket_kernel()(srcp, dstp)

    h1s, ab1 = _tc1(x_pad, W1.astype(_f32), a1)
    absrc1 = ab1[:H1].reshape(H1 * NP)
    abdst1 = ab1[H1:].reshape(H1 * NP)
    h1f = h1s.reshape(2 * NP, 2 * DH)
    acc1a, den1a = _edge_kernel(2 * DH, 0)(bs0, bd0, cnt0, h1f,
                                           absrc1, abdst1, z1, zd)
    acc1b, den1b = _edge_kernel(2 * DH, 1)(bs1, bd1, cnt1, h1f,
                                           absrc1, abdst1, z1, zd)
    acc1 = jnp.concatenate([acc1a, acc1b], axis=1)
    den1 = jnp.concatenate([den1a, den1b], axis=1)

    h2s, ab2 = _tc2(acc1, den1, b1.astype(_f32).reshape(1, H1 * DH),
                    W2.astype(_f32), a2)
    absrc2 = jnp.broadcast_to(ab2[0][None, :], (H1, NP)).reshape(H1 * NP)
    abdst2 = jnp.broadcast_to(ab2[1][None, :], (H1, NP)).reshape(H1 * NP)
    h2f = h2s.reshape(2 * NP, DO // 2)
    acc2a, den2a = _edge_kernel(DO // 2, 0)(bs0, bd0, cnt0, h2f,
                                            absrc2, abdst2, z2, zd)
    acc2b, den2b = _edge_kernel(DO // 2, 1)(bs1, bd1, cnt1, h2f,
                                            absrc2, abdst2, z2, zd)
    acc2 = jnp.concatenate([acc2a, acc2b], axis=1)
    den2 = jnp.concatenate([den2a, den2b], axis=1)

    out = _tc3(acc2, den2, b2.astype(_f32).reshape(1, DO))
    return out[:N]
